# baseline (speedup 1.0000x reference)
import functools

import jax
import jax.numpy as jnp
from jax import lax
from jax.experimental import pallas as pl
from jax.experimental.pallas import tpu as pltpu
from jax.experimental.pallas import tpu_sc as plsc

NUM_CHANNELS = 96
NUM_PIXELS = 512 * 512
NUM_SAMPLES = 26214

NC = 2
NS = 16
CPC = NUM_CHANNELS // NC
TPW = 1640
S_PAD = NS * TPW
SEG = NUM_PIXELS // NS
NSLOT = 3


def _sc_gather(img_flat_hbm, idxp_hbm):
    mesh = plsc.VectorSubcoreMesh(core_axis_name="c", subcore_axis_name="s")

    @functools.partial(
        pl.kernel,
        mesh=mesh,
        out_type=jax.ShapeDtypeStruct((NUM_CHANNELS * S_PAD,), jnp.float32),
        scratch_types=[
            pltpu.VMEM((TPW,), jnp.int32),
            pltpu.VMEM((CPC * TPW,), jnp.float32),
            pltpu.VMEM_SHARED((NSLOT * NUM_PIXELS,), jnp.float32),
            pltpu.SemaphoreType.DMA,
            pltpu.SemaphoreType.DMA,
            pltpu.SemaphoreType.DMA,
            pltpu.SemaphoreType.DMA,
            pltpu.SemaphoreType.DMA,
        ],
    )
    def k(img_hbm, idx_hbm, out_hbm, idx_v, res_v, rows_s,
          sem_s0, sem_s1, sem_g0, sem_g1, sem_w):
        cid = lax.axis_index("c")
        sid = lax.axis_index("s")
        c0 = cid * CPC
        pltpu.sync_copy(idx_hbm.at[pl.ds(sid * TPW, TPW)], idx_v)

        def stage(k_next, sem):
            slot = lax.rem(k_next, NSLOT)
            src = pl.ds(pl.multiple_of((c0 + k_next) * NUM_PIXELS + sid * SEG, 8),
                        SEG)
            dst = pl.ds(pl.multiple_of(slot * NUM_PIXELS + sid * SEG, 8), SEG)
            pltpu.async_copy(img_hbm.at[src], rows_s.at[dst], sem)

        def wait_stage(sem):
            pltpu.make_async_copy(img_hbm.at[pl.ds(0, SEG)],
                                  rows_s.at[pl.ds(0, SEG)], sem).wait()

        def fire_gather(kk, sem):
            slot = lax.rem(kk, NSLOT)
            src = rows_s.at[pl.ds(pl.multiple_of(slot * NUM_PIXELS, 8),
                                  NUM_PIXELS)]
            for (o, n) in ((0, 416), (416, 408), (824, 408), (1232, 408)):
                dst = res_v.at[pl.ds(kk * TPW + o, n)]
                pltpu.async_copy(src.at[idx_v.at[pl.ds(o, n)]], dst, sem)

        def wait_gather(sem):
            pltpu.make_async_copy(img_hbm.at[pl.ds(0, TPW)],
                                  res_v.at[pl.ds(0, TPW)], sem).wait()

        def fire_write(kk):
            dst = pl.ds(pl.multiple_of((c0 + kk) * S_PAD + sid * TPW, 8), TPW)
            pltpu.async_copy(res_v.at[pl.ds(kk * TPW, TPW)],
                             out_hbm.at[dst], sem_w)

        stage(0, sem_s0)
        stage(1, sem_s1)

        def chan(kk, carry):
            even = lax.rem(kk, 2) == 0

            @pl.when(even)
            def _():
                wait_stage(sem_s0)

            @pl.when(jnp.logical_not(even))
            def _():
                wait_stage(sem_s1)

            plsc.subcore_barrier()

            @pl.when(even)
            def _():
                fire_gather(kk, sem_g0)

            @pl.when(jnp.logical_not(even))
            def _():
                fire_gather(kk, sem_g1)

            @pl.when(jnp.logical_and(kk >= 1, jnp.logical_not(even)))
            def _():
                wait_gather(sem_g0)

            @pl.when(jnp.logical_and(kk >= 1, even))
            def _():
                wait_gather(sem_g1)

            @pl.when(kk >= 1)
            def _():
                fire_write(kk - 1)

            plsc.subcore_barrier()

            @pl.when(jnp.logical_and(even, kk < CPC - 2))
            def _():
                stage(kk + 2, sem_s0)

            @pl.when(jnp.logical_and(jnp.logical_not(even), kk < CPC - 2))
            def _():
                stage(kk + 2, sem_s1)

            return carry

        lax.fori_loop(0, CPC, chan, 0)

        wait_gather(sem_g1)
        fire_write(CPC - 1)
        pltpu.make_async_copy(img_hbm.at[pl.ds(0, CPC * TPW)],
                              res_v, sem_w).wait()

    return k(img_flat_hbm, idxp_hbm)


def kernel(image_flat, idx):
    idx_i = idx.astype(jnp.int32)
    idxp = jnp.pad(idx_i, (0, S_PAD - NUM_SAMPLES))
    out_flat = _sc_gather(image_flat.reshape(-1), idxp)
    image_subset = out_flat.reshape(NUM_CHANNELS, S_PAD)[:, :NUM_SAMPLES]
    return (image_subset, idx)
